# baseline (speedup 1.0000x reference)
import jax
import jax.numpy as jnp
from jax.experimental import pallas as pl

_BM = 400
_H = 256
_G = 16

_HI = jax.lax.Precision.HIGHEST


def _virt_kernel(virt, wvirt, bvirt, vrow_o):
    vrow_o[...] = jnp.dot(virt[...], wvirt[...],
                          preferred_element_type=jnp.float32) + bvirt[...]


def _embed_kernel(esm, dssp, vrow, wdssp, bdssp, wemb, bemb, o):
    dssp_c = dssp[...] + vrow[...]
    proj = jnp.dot(dssp_c, wdssp[...], preferred_element_type=jnp.float32)
    x = esm[...] + proj + bdssp[...]
    o[...] = jnp.maximum(
        jnp.dot(x, wemb[...], preferred_element_type=jnp.float32) + bemb[...], 0.0)


def _gatpre_kernel(x, wg, As, Ad, h_o, as_o, ad_o):
    h = jnp.dot(x[...], wg[...], preferred_element_type=jnp.float32)
    h_o[...] = h
    as_o[...] = jnp.dot(h, As[...], preferred_element_type=jnp.float32, precision=_HI)
    ad_o[...] = jnp.dot(h, Ad[...], preferred_element_type=jnp.float32, precision=_HI)


def _post_kernel(m, res, bg, g, b, y_o):
    t = m[...] + bg[...] + res[...]
    mu = jnp.mean(t, axis=1, keepdims=True)
    var = jnp.mean((t - mu) ** 2, axis=1, keepdims=True)
    y = (t - mu) / jnp.sqrt(var + 1e-5) * g[...] + b[...]
    y_o[...] = jnp.maximum(y, 0.0)


def _pool_kernel(x, batch, sums_o, cnt_o):
    i = pl.program_id(0)

    @pl.when(i == 0)
    def _():
        sums_o[...] = jnp.zeros_like(sums_o)
        cnt_o[...] = jnp.zeros_like(cnt_o)

    ids = batch[...]
    iota = jax.lax.broadcasted_iota(jnp.int32, (ids.shape[0], _G), 1)
    mask = (ids == iota).astype(jnp.float32)
    sums_o[...] += jax.lax.dot_general(
        mask, x[...], (((0,), (0,)), ((), ())),
        preferred_element_type=jnp.float32, precision=_HI)
    cnt_o[...] += jnp.sum(mask, axis=0)[:, None]


def _mlp_kernel(sums, cnt, w1, b1, w2, b2, o):
    g = sums[...] / jnp.maximum(cnt[:, :1], 1.0)
    t = jnp.maximum(
        jnp.dot(g, w1[...], preferred_element_type=jnp.float32) + b1[...], 0.0)
    o[...] = jnp.dot(t, w2[...], preferred_element_type=jnp.float32) + b2[...]


def _row(v):
    return v.reshape(1, -1)


def _mk_attn_mat(a, heads):
    blocks = [a[k][:, None] for k in range(heads)]
    A = jax.scipy.linalg.block_diag(*blocks)
    return jnp.pad(A, ((0, 0), (0, 128 - heads)))


def _gat_layer(x, src, dst, W, a_s, a_d, heads):
    n = x.shape[0]
    grid = n // _BM
    dh = W.shape[1]
    As = _mk_attn_mat(a_s, heads)
    Ad = _mk_attn_mat(a_d, heads)
    h, asv, adv = pl.pallas_call(
        _gatpre_kernel,
        grid=(grid,),
        in_specs=[
            pl.BlockSpec((_BM, _H), lambda i: (i, 0)),
            pl.BlockSpec((_H, dh), lambda i: (0, 0)),
            pl.BlockSpec((dh, 128), lambda i: (0, 0)),
            pl.BlockSpec((dh, 128), lambda i: (0, 0)),
        ],
        out_specs=[
            pl.BlockSpec((_BM, dh), lambda i: (i, 0)),
            pl.BlockSpec((_BM, 128), lambda i: (i, 0)),
            pl.BlockSpec((_BM, 128), lambda i: (i, 0)),
        ],
        out_shape=[
            jax.ShapeDtypeStruct((n, dh), jnp.float32),
            jax.ShapeDtypeStruct((n, 128), jnp.float32),
            jax.ShapeDtypeStruct((n, 128), jnp.float32),
        ],
    )(x, W, As, Ad)
    asv = asv[:, :heads]
    adv = adv[:, :heads]

    e = jax.nn.leaky_relu(asv[src] + adv[dst], 0.2)
    emax = jax.ops.segment_max(e, dst, num_segments=n)
    emax = jnp.where(jnp.isneginf(emax), 0.0, emax)
    ee = jnp.exp(e - emax[dst])
    den = jax.ops.segment_sum(ee, dst, num_segments=n)
    alpha = ee / (den[dst] + 1e-16)
    h3 = h.reshape(n, heads, _H)
    out = jax.ops.segment_sum(h3[src] * alpha[:, :, None], dst, num_segments=n)
    return out.mean(axis=1)


def _post(m, res, bg, g, b):
    n = m.shape[0]
    grid = n // _BM
    return pl.pallas_call(
        _post_kernel,
        grid=(grid,),
        in_specs=[
            pl.BlockSpec((_BM, _H), lambda i: (i, 0)),
            pl.BlockSpec((_BM, _H), lambda i: (i, 0)),
            pl.BlockSpec((1, _H), lambda i: (0, 0)),
            pl.BlockSpec((1, _H), lambda i: (0, 0)),
            pl.BlockSpec((1, _H), lambda i: (0, 0)),
        ],
        out_specs=pl.BlockSpec((_BM, _H), lambda i: (i, 0)),
        out_shape=jax.ShapeDtypeStruct((n, _H), jnp.float32),
    )(m, res, _row(bg), _row(g), _row(b))


def kernel(esm, dssp, virtual, edge_index, batch, W_emb, b_emb, W_virt, b_virt,
           W_dssp, b_dssp, W_g1, a_src1, a_dst1, b_g1, g_ln1, b_ln1,
           W_g2, a_src2, a_dst2, b_g2, g_ln2, b_ln2,
           W_g3, a_src3, a_dst3, b_g3, g_ln3, b_ln3,
           W_m1, b_m1, W_m2, b_m2):
    n = esm.shape[0]
    d_in = esm.shape[1]
    d_dssp = dssp.shape[1]
    grid = n // _BM

    vrow = pl.pallas_call(
        _virt_kernel,
        out_shape=jax.ShapeDtypeStruct((1, d_dssp), jnp.float32),
    )(virtual, W_virt, _row(b_virt))

    dssp_p = jnp.pad(dssp, ((0, 0), (0, 128 - d_dssp)))
    vrow_p = jnp.pad(vrow, ((0, 0), (0, 128 - d_dssp)))
    wdssp_p = jnp.pad(W_dssp, ((0, 128 - d_dssp), (0, 0)))

    x0 = pl.pallas_call(
        _embed_kernel,
        grid=(grid,),
        in_specs=[
            pl.BlockSpec((_BM, d_in), lambda i: (i, 0)),
            pl.BlockSpec((_BM, 128), lambda i: (i, 0)),
            pl.BlockSpec((1, 128), lambda i: (0, 0)),
            pl.BlockSpec((128, d_in), lambda i: (0, 0)),
            pl.BlockSpec((1, d_in), lambda i: (0, 0)),
            pl.BlockSpec((d_in, _H), lambda i: (0, 0)),
            pl.BlockSpec((1, _H), lambda i: (0, 0)),
        ],
        out_specs=pl.BlockSpec((_BM, _H), lambda i: (i, 0)),
        out_shape=jax.ShapeDtypeStruct((n, _H), jnp.float32),
    )(esm, dssp_p, vrow_p, wdssp_p, _row(b_dssp), W_emb, _row(b_emb))

    loop = jnp.arange(n, dtype=edge_index.dtype)
    src = jnp.concatenate([edge_index[0], loop])
    dst = jnp.concatenate([edge_index[1], loop])

    x = x0
    m1 = _gat_layer(x, src, dst, W_g1, a_src1, a_dst1, 4)
    x = _post(m1, x, b_g1, g_ln1, b_ln1)
    m2 = _gat_layer(x, src, dst, W_g2, a_src2, a_dst2, 4)
    x = _post(m2, x, b_g2, g_ln2, b_ln2)
    m3 = _gat_layer(x, src, dst, W_g3, a_src3, a_dst3, 1)
    x = _post(m3, x, b_g3, g_ln3, b_ln3)

    batch2 = batch.reshape(n, 1)
    sums, cnt = pl.pallas_call(
        _pool_kernel,
        grid=(grid,),
        in_specs=[
            pl.BlockSpec((_BM, _H), lambda i: (i, 0)),
            pl.BlockSpec((_BM, 1), lambda i: (i, 0)),
        ],
        out_specs=[
            pl.BlockSpec((_G, _H), lambda i: (0, 0)),
            pl.BlockSpec((_G, 128), lambda i: (0, 0)),
        ],
        out_shape=[
            jax.ShapeDtypeStruct((_G, _H), jnp.float32),
            jax.ShapeDtypeStruct((_G, 128), jnp.float32),
        ],
    )(x, batch2)

    enhanced = pl.pallas_call(
        _mlp_kernel,
        out_shape=jax.ShapeDtypeStruct((_G, _H), jnp.float32),
    )(sums, cnt, W_m1, _row(b_m1), W_m2, _row(b_m2))

    return (enhanced, x)


# speedup vs baseline: 1.0006x; 1.0006x over previous
import jax
import jax.numpy as jnp
from jax.experimental import pallas as pl

_BM = 400
_H = 256
_G = 16

_HI = jax.lax.Precision.HIGHEST


def _virt_kernel(virt, wvirt, bvirt, vrow_o):
    vrow_o[...] = jnp.dot(virt[...], wvirt[...],
                          preferred_element_type=jnp.float32) + bvirt[...]


def _embed_kernel(esm, dssp, vrow, wdssp, bdssp, wemb, bemb, o):
    dssp_c = dssp[...] + vrow[...]
    proj = jnp.dot(dssp_c, wdssp[...], preferred_element_type=jnp.float32)
    x = esm[...] + proj + bdssp[...]
    o[...] = jnp.maximum(
        jnp.dot(x, wemb[...], preferred_element_type=jnp.float32) + bemb[...], 0.0)


def _gatpre_kernel(x, wg, As, Ad, h_o, as_o, ad_o):
    h = jnp.dot(x[...], wg[...], preferred_element_type=jnp.float32)
    h_o[...] = h
    as_o[...] = jnp.dot(h, As[...], preferred_element_type=jnp.float32, precision=_HI)
    ad_o[...] = jnp.dot(h, Ad[...], preferred_element_type=jnp.float32, precision=_HI)


def _gatfused_kernel(m, res, bg, g, b, wg, As, Ad, y_o, h_o, as_o, ad_o):
    t = m[...] + bg[...] + res[...]
    mu = jnp.mean(t, axis=1, keepdims=True)
    var = jnp.mean((t - mu) ** 2, axis=1, keepdims=True)
    y = jnp.maximum((t - mu) / jnp.sqrt(var + 1e-5) * g[...] + b[...], 0.0)
    y_o[...] = y
    h = jnp.dot(y, wg[...], preferred_element_type=jnp.float32)
    h_o[...] = h
    as_o[...] = jnp.dot(h, As[...], preferred_element_type=jnp.float32, precision=_HI)
    ad_o[...] = jnp.dot(h, Ad[...], preferred_element_type=jnp.float32, precision=_HI)


def _post_kernel(m, res, bg, g, b, y_o):
    t = m[...] + bg[...] + res[...]
    mu = jnp.mean(t, axis=1, keepdims=True)
    var = jnp.mean((t - mu) ** 2, axis=1, keepdims=True)
    y = (t - mu) / jnp.sqrt(var + 1e-5) * g[...] + b[...]
    y_o[...] = jnp.maximum(y, 0.0)


def _pool_kernel(x, batch, sums_o, cnt_o):
    i = pl.program_id(0)

    @pl.when(i == 0)
    def _():
        sums_o[...] = jnp.zeros_like(sums_o)
        cnt_o[...] = jnp.zeros_like(cnt_o)

    ids = batch[...]
    iota = jax.lax.broadcasted_iota(jnp.int32, (ids.shape[0], _G), 1)
    mask = (ids == iota).astype(jnp.float32)
    sums_o[...] += jax.lax.dot_general(
        mask, x[...], (((0,), (0,)), ((), ())),
        preferred_element_type=jnp.float32, precision=_HI)
    cnt_o[...] += jnp.sum(mask, axis=0)[:, None]


def _mlp_kernel(sums, cnt, w1, b1, w2, b2, o):
    g = sums[...] / jnp.maximum(cnt[:, :1], 1.0)
    t = jnp.maximum(
        jnp.dot(g, w1[...], preferred_element_type=jnp.float32) + b1[...], 0.0)
    o[...] = jnp.dot(t, w2[...], preferred_element_type=jnp.float32) + b2[...]


def _row(v):
    return v.reshape(1, -1)


def _mk_attn_mat(a, heads):
    blocks = [a[k][:, None] for k in range(heads)]
    A = jax.scipy.linalg.block_diag(*blocks)
    return jnp.pad(A, ((0, 0), (0, 128 - heads)))


def _edge_softmax(h, asv, adv, src, dst, heads):
    n = h.shape[0]
    asv = asv[:, :heads]
    adv = adv[:, :heads]
    e = jax.nn.leaky_relu(asv[src] + adv[dst], 0.2)
    emax = jax.ops.segment_max(e, dst, num_segments=n)
    emax = jnp.where(jnp.isneginf(emax), 0.0, emax)
    ee = jnp.exp(e - emax[dst])
    den = jax.ops.segment_sum(ee, dst, num_segments=n)
    alpha = ee / (den[dst] + 1e-16)
    h3 = h.reshape(n, heads, _H)
    out = jax.ops.segment_sum(h3[src] * alpha[:, :, None], dst, num_segments=n)
    return out.mean(axis=1)


def _gat_fused(m, res, bg, g, b, W, a_s, a_d, heads):
    n = m.shape[0]
    grid = n // _BM
    dh = W.shape[1]
    As = _mk_attn_mat(a_s, heads)
    Ad = _mk_attn_mat(a_d, heads)
    return pl.pallas_call(
        _gatfused_kernel,
        grid=(grid,),
        in_specs=[
            pl.BlockSpec((_BM, _H), lambda i: (i, 0)),
            pl.BlockSpec((_BM, _H), lambda i: (i, 0)),
            pl.BlockSpec((1, _H), lambda i: (0, 0)),
            pl.BlockSpec((1, _H), lambda i: (0, 0)),
            pl.BlockSpec((1, _H), lambda i: (0, 0)),
            pl.BlockSpec((_H, dh), lambda i: (0, 0)),
            pl.BlockSpec((dh, 128), lambda i: (0, 0)),
            pl.BlockSpec((dh, 128), lambda i: (0, 0)),
        ],
        out_specs=[
            pl.BlockSpec((_BM, _H), lambda i: (i, 0)),
            pl.BlockSpec((_BM, dh), lambda i: (i, 0)),
            pl.BlockSpec((_BM, 128), lambda i: (i, 0)),
            pl.BlockSpec((_BM, 128), lambda i: (i, 0)),
        ],
        out_shape=[
            jax.ShapeDtypeStruct((n, _H), jnp.float32),
            jax.ShapeDtypeStruct((n, dh), jnp.float32),
            jax.ShapeDtypeStruct((n, 128), jnp.float32),
            jax.ShapeDtypeStruct((n, 128), jnp.float32),
        ],
    )(m, res, _row(bg), _row(g), _row(b), W, As, Ad)


def _gat_pre(x, W, a_s, a_d, heads):
    n = x.shape[0]
    grid = n // _BM
    dh = W.shape[1]
    As = _mk_attn_mat(a_s, heads)
    Ad = _mk_attn_mat(a_d, heads)
    return pl.pallas_call(
        _gatpre_kernel,
        grid=(grid,),
        in_specs=[
            pl.BlockSpec((_BM, _H), lambda i: (i, 0)),
            pl.BlockSpec((_H, dh), lambda i: (0, 0)),
            pl.BlockSpec((dh, 128), lambda i: (0, 0)),
            pl.BlockSpec((dh, 128), lambda i: (0, 0)),
        ],
        out_specs=[
            pl.BlockSpec((_BM, dh), lambda i: (i, 0)),
            pl.BlockSpec((_BM, 128), lambda i: (i, 0)),
            pl.BlockSpec((_BM, 128), lambda i: (i, 0)),
        ],
        out_shape=[
            jax.ShapeDtypeStruct((n, dh), jnp.float32),
            jax.ShapeDtypeStruct((n, 128), jnp.float32),
            jax.ShapeDtypeStruct((n, 128), jnp.float32),
        ],
    )(x, W, As, Ad)


def _post(m, res, bg, g, b):
    n = m.shape[0]
    grid = n // _BM
    return pl.pallas_call(
        _post_kernel,
        grid=(grid,),
        in_specs=[
            pl.BlockSpec((_BM, _H), lambda i: (i, 0)),
            pl.BlockSpec((_BM, _H), lambda i: (i, 0)),
            pl.BlockSpec((1, _H), lambda i: (0, 0)),
            pl.BlockSpec((1, _H), lambda i: (0, 0)),
            pl.BlockSpec((1, _H), lambda i: (0, 0)),
        ],
        out_specs=pl.BlockSpec((_BM, _H), lambda i: (i, 0)),
        out_shape=jax.ShapeDtypeStruct((n, _H), jnp.float32),
    )(m, res, _row(bg), _row(g), _row(b))


def kernel(esm, dssp, virtual, edge_index, batch, W_emb, b_emb, W_virt, b_virt,
           W_dssp, b_dssp, W_g1, a_src1, a_dst1, b_g1, g_ln1, b_ln1,
           W_g2, a_src2, a_dst2, b_g2, g_ln2, b_ln2,
           W_g3, a_src3, a_dst3, b_g3, g_ln3, b_ln3,
           W_m1, b_m1, W_m2, b_m2):
    n = esm.shape[0]
    d_in = esm.shape[1]
    d_dssp = dssp.shape[1]
    grid = n // _BM

    vrow = pl.pallas_call(
        _virt_kernel,
        out_shape=jax.ShapeDtypeStruct((1, d_dssp), jnp.float32),
    )(virtual, W_virt, _row(b_virt))

    dssp_p = jnp.pad(dssp, ((0, 0), (0, 128 - d_dssp)))
    vrow_p = jnp.pad(vrow, ((0, 0), (0, 128 - d_dssp)))
    wdssp_p = jnp.pad(W_dssp, ((0, 128 - d_dssp), (0, 0)))

    x0 = pl.pallas_call(
        _embed_kernel,
        grid=(grid,),
        in_specs=[
            pl.BlockSpec((_BM, d_in), lambda i: (i, 0)),
            pl.BlockSpec((_BM, 128), lambda i: (i, 0)),
            pl.BlockSpec((1, 128), lambda i: (0, 0)),
            pl.BlockSpec((128, d_in), lambda i: (0, 0)),
            pl.BlockSpec((1, d_in), lambda i: (0, 0)),
            pl.BlockSpec((d_in, _H), lambda i: (0, 0)),
            pl.BlockSpec((1, _H), lambda i: (0, 0)),
        ],
        out_specs=pl.BlockSpec((_BM, _H), lambda i: (i, 0)),
        out_shape=jax.ShapeDtypeStruct((n, _H), jnp.float32),
    )(esm, dssp_p, vrow_p, wdssp_p, _row(b_dssp), W_emb, _row(b_emb))

    loop = jnp.arange(n, dtype=edge_index.dtype)
    src = jnp.concatenate([edge_index[0], loop])
    dst = jnp.concatenate([edge_index[1], loop])

    h1, as1, ad1 = _gat_pre(x0, W_g1, a_src1, a_dst1, 4)
    m1 = _edge_softmax(h1, as1, ad1, src, dst, 4)
    y1, h2, as2, ad2 = _gat_fused(m1, x0, b_g1, g_ln1, b_ln1, W_g2, a_src2, a_dst2, 4)
    m2 = _edge_softmax(h2, as2, ad2, src, dst, 4)
    y2, h3, as3, ad3 = _gat_fused(m2, y1, b_g2, g_ln2, b_ln2, W_g3, a_src3, a_dst3, 1)
    m3 = _edge_softmax(h3, as3, ad3, src, dst, 1)
    x = _post(m3, y2, b_g3, g_ln3, b_ln3)

    batch2 = batch.reshape(n, 1)
    sums, cnt = pl.pallas_call(
        _pool_kernel,
        grid=(grid,),
        in_specs=[
            pl.BlockSpec((_BM, _H), lambda i: (i, 0)),
            pl.BlockSpec((_BM, 1), lambda i: (i, 0)),
        ],
        out_specs=[
            pl.BlockSpec((_G, _H), lambda i: (0, 0)),
            pl.BlockSpec((_G, 128), lambda i: (0, 0)),
        ],
        out_shape=[
            jax.ShapeDtypeStruct((_G, _H), jnp.float32),
            jax.ShapeDtypeStruct((_G, 128), jnp.float32),
        ],
    )(x, batch2)

    enhanced = pl.pallas_call(
        _mlp_kernel,
        out_shape=jax.ShapeDtypeStruct((_G, _H), jnp.float32),
    )(sums, cnt, W_m1, _row(b_m1), W_m2, _row(b_m2))

    return (enhanced, x)
